# Initial kernel scaffold; baseline (speedup 1.0000x reference)
import jax
import jax.numpy as jnp
from jax.experimental import pallas as pl


def kernel(x, edge_index, W, b):
    raise NotImplementedError("write your pallas kernel here")


# speedup vs baseline: 60.7008x; 60.7008x over previous
import functools

import jax
import jax.numpy as jnp
from jax import lax
from jax.experimental import pallas as pl
from jax.experimental.pallas import tpu as pltpu
from jax.experimental.pallas import tpu_sc as plsc

NC = 2
NS = 16
NW = NC * NS
LANES = 16


def _round_up(v, m):
    return (v + m - 1) // m * m


@functools.lru_cache(maxsize=None)
def _make_hist(ept, np_):
    mesh = plsc.VectorSubcoreMesh(core_axis_name="c", subcore_axis_name="s", num_cores=NC, num_subcores=NS)

    @functools.partial(
        pl.kernel,
        out_type=jax.ShapeDtypeStruct((NW, np_), jnp.float32),
        mesh=mesh,
        scratch_types=[
            pltpu.VMEM((ept,), jnp.int32),
            pltpu.VMEM((np_,), jnp.float32),
        ],
        compiler_params=pltpu.CompilerParams(needs_layout_passes=False),
    )
    def hist(dst_hbm, deg_hbm, idx_v, hist_v):
        wid = lax.axis_index("s") * NC + lax.axis_index("c")

        def zero(i, c):
            hist_v[pl.ds(i * LANES, LANES)] = jnp.zeros((LANES,), jnp.float32)
            return c

        lax.fori_loop(0, np_ // LANES, zero, 0)
        pltpu.sync_copy(dst_hbm.at[pl.ds(wid * ept, ept)], idx_v)
        ones = jnp.ones((LANES,), jnp.float32)

        def body(i, c):
            idx = idx_v[pl.ds(i * LANES, LANES)]
            plsc.addupdate_scatter(hist_v, [idx], ones)
            return c

        lax.fori_loop(0, ept // LANES, body, 0)
        pltpu.sync_copy(hist_v, deg_hbm.at[wid])

    return hist


@functools.lru_cache(maxsize=None)
def _make_gather_scatter(ept, np_):
    mesh = plsc.VectorSubcoreMesh(core_axis_name="c", subcore_axis_name="s", num_cores=NC, num_subcores=NS)

    @functools.partial(
        pl.kernel,
        out_type=jax.ShapeDtypeStruct((NW, np_), jnp.float32),
        mesh=mesh,
        scratch_types=[
            pltpu.VMEM((ept,), jnp.int32),
            pltpu.VMEM((ept,), jnp.int32),
            pltpu.VMEM((np_,), jnp.float32),
            pltpu.VMEM((np_,), jnp.float32),
        ],
        compiler_params=pltpu.CompilerParams(needs_layout_passes=False),
    )
    def gs(src_hbm, dst_hbm, z_hbm, acc_hbm, src_v, dst_v, z_v, acc_v):
        wid = lax.axis_index("s") * NC + lax.axis_index("c")

        def zero(i, c):
            acc_v[pl.ds(i * LANES, LANES)] = jnp.zeros((LANES,), jnp.float32)
            return c

        lax.fori_loop(0, np_ // LANES, zero, 0)
        pltpu.sync_copy(z_hbm, z_v)
        pltpu.sync_copy(src_hbm.at[pl.ds(wid * ept, ept)], src_v)
        pltpu.sync_copy(dst_hbm.at[pl.ds(wid * ept, ept)], dst_v)

        def body(i, c):
            s = src_v[pl.ds(i * LANES, LANES)]
            d = dst_v[pl.ds(i * LANES, LANES)]
            vals = plsc.load_gather(z_v, [s])
            plsc.addupdate_scatter(acc_v, [d], vals)
            return c

        lax.fori_loop(0, ept // LANES, body, 0)
        pltpu.sync_copy(acc_v, acc_hbm.at[wid])

    return gs


def _t1_body(x_ref, wt_ref, degp_ref, z_ref, dinv_ref):
    y = lax.dot_general(
        wt_ref[...], x_ref[...],
        dimension_numbers=(((1,), (1,)), ((), ())),
        preferred_element_type=jnp.float32,
    )
    deg = jnp.sum(degp_ref[...], axis=0) + 1.0
    dinv = lax.rsqrt(deg)
    dinv_ref[...] = dinv
    z_ref[...] = dinv * y[0]


def _f_body(z_ref, dinv_ref, accp_ref, b_ref, out_ref):
    acc = jnp.sum(accp_ref[...], axis=0)
    out_ref[...] = dinv_ref[...] * (z_ref[...] + acc) + b_ref[...]


def kernel(x, edge_index, W, b):
    n, d_in = x.shape
    e = edge_index.shape[1]
    BL = 2048
    np_ = _round_up(n, BL)
    gr = np_ // BL
    ept = _round_up(-(-e // NW), LANES)
    ep = ept * NW

    src = edge_index[0]
    dst = edge_index[1]
    pad_e = ep - e
    pad_idx = jnp.full((pad_e,), n, dtype=jnp.int32)
    src_p = jnp.concatenate([src, pad_idx])
    dst_p = jnp.concatenate([dst, pad_idx])
    x_p = jnp.pad(x, ((0, np_ - n), (0, 0)))
    wt = W.T
    b1 = b.reshape(1)

    degp = _make_hist(ept, np_)(dst_p)

    z, dinv = pl.pallas_call(
        _t1_body,
        grid=(gr,),
        in_specs=[
            pl.BlockSpec((BL, d_in), lambda i: (i, 0)),
            pl.BlockSpec((1, d_in), lambda i: (0, 0)),
            pl.BlockSpec((NW, BL), lambda i: (0, i)),
        ],
        out_specs=[
            pl.BlockSpec((BL,), lambda i: (i,)),
            pl.BlockSpec((BL,), lambda i: (i,)),
        ],
        out_shape=[jax.ShapeDtypeStruct((np_,), jnp.float32)] * 2,
    )(x_p, wt, degp)

    accp = _make_gather_scatter(ept, np_)(src_p, dst_p, z)

    out = pl.pallas_call(
        _f_body,
        grid=(gr,),
        in_specs=[
            pl.BlockSpec((BL,), lambda i: (i,)),
            pl.BlockSpec((BL,), lambda i: (i,)),
            pl.BlockSpec((NW, BL), lambda i: (0, i)),
            pl.BlockSpec((1,), lambda i: (0,)),
        ],
        out_specs=pl.BlockSpec((BL,), lambda i: (i,)),
        out_shape=jax.ShapeDtypeStruct((np_,), jnp.float32),
    )(z, dinv, accp, b1)

    return out[:n].reshape(n, 1)
